# baseline (speedup 1.0000x reference)
import functools

import jax
import jax.numpy as jnp
from jax import lax
from jax.experimental import pallas as pl
from jax.experimental.pallas import tpu as pltpu
from jax.experimental.pallas import tpu_sc as plsc

B = 16384
K = 32
NC = 2
NS = 16
NW = NC * NS
BPW = B // NW
CHUNK = 128
L = 16
HALF = 256
NPASS = BPW // HALF
ROWS = 4 * K


def _sc_body(uidx_hbm, iidx_hbm, utab_hbm, itab_hbm, out_hbm,
             uidx_v, iidx_v, urow_v, irow_v, ubuf, ibuf, out_v, sem):
    c = lax.axis_index("c")
    s = lax.axis_index("s")
    wid = s * NC + c
    base = wid * BPW

    pltpu.sync_copy(uidx_hbm.at[pl.ds(base, BPW)], uidx_v)
    pltpu.sync_copy(iidx_hbm.at[pl.ds(base, BPW)], iidx_v)

    def rowprep(t, carry):
        o = t * L
        urow_v[pl.ds(o, L)] = lax.shift_right_logical(uidx_v[pl.ds(o, L)], 2)
        irow_v[pl.ds(o, L)] = lax.shift_right_logical(iidx_v[pl.ds(o, L)], 2)
        return carry

    lax.fori_loop(0, BPW // L, rowprep, 0)

    lanes = lax.iota(jnp.int32, 16)

    for p in range(NPASS):
        copies = []
        for j in range(HALF // CHUNK):
            o = p * HALF + j * CHUNK
            cu = pltpu.make_async_copy(
                utab_hbm.at[urow_v.at[pl.ds(o, CHUNK)]],
                ubuf.at[pl.ds(j * CHUNK, CHUNK)], sem)
            ci = pltpu.make_async_copy(
                itab_hbm.at[irow_v.at[pl.ds(o, CHUNK)]],
                ibuf.at[pl.ds(j * CHUNK, CHUNK)], sem)
            cu.start()
            ci.start()
            copies.append(cu)
            copies.append(ci)
        for cp in copies:
            cp.wait()

        def group(g, carry, p=p):
            o = p * HALF + g * L
            slot = g * L + lanes
            ucol = (uidx_v[pl.ds(o, L)] & 3) * K
            icol = (iidx_v[pl.ds(o, L)] & 3) * K
            acc = jnp.zeros((L,), jnp.float32)
            for k in range(K):
                u = plsc.load_gather(ubuf, [slot, ucol + k])
                v = plsc.load_gather(ibuf, [slot, icol + k])
                acc = acc + u * v
            out_v[pl.ds(o, L)] = acc
            return carry

        lax.fori_loop(0, HALF // L, group, 0)

    pltpu.sync_copy(out_v, out_hbm.at[pl.ds(base, BPW)])


_sc_call = functools.partial(
    pl.kernel,
    out_type=jax.ShapeDtypeStruct((B,), jnp.float32),
    mesh=plsc.VectorSubcoreMesh(core_axis_name="c", subcore_axis_name="s"),
    scratch_types=[
        pltpu.VMEM((BPW,), jnp.int32),
        pltpu.VMEM((BPW,), jnp.int32),
        pltpu.VMEM((BPW,), jnp.int32),
        pltpu.VMEM((BPW,), jnp.int32),
        pltpu.VMEM((HALF, ROWS), jnp.float32),
        pltpu.VMEM((HALF, ROWS), jnp.float32),
        pltpu.VMEM((BPW,), jnp.float32),
        pltpu.SemaphoreType.DMA,
    ],
    compiler_params=pltpu.CompilerParams(needs_layout_passes=False),
)(_sc_body)


def kernel(user_input, item_input, user_embedding, item_embedding):
    utab = user_embedding.reshape(250000, ROWS)
    itab = item_embedding.reshape(250000, ROWS)
    out = _sc_call(user_input.reshape(B), item_input.reshape(B), utab, itab)
    return out.reshape(B, 1)
